# Initial kernel scaffold; baseline (speedup 1.0000x reference)
import jax
import jax.numpy as jnp
from jax.experimental import pallas as pl


def kernel(indices, embeddings):
    raise NotImplementedError("write your pallas kernel here")


# speedup vs baseline: 1.5973x; 1.5973x over previous
import functools

import jax
import jax.numpy as jnp
from jax import lax
from jax.experimental import pallas as pl
from jax.experimental.pallas import tpu as pltpu
from jax.experimental.pallas import tpu_sc as plsc

BATCH = 4096
TOKENS = 200
DIM = 2048
ROWS = BATCH * TOKENS

NUM_CORES = 2
NUM_SUBCORES = 16
NUM_WORKERS = NUM_CORES * NUM_SUBCORES
PER_WORKER = ROWS // NUM_WORKERS

CHUNK = 16
NBUF = 2
NCHUNKS = PER_WORKER // CHUNK


def _sc_body(idx_hbm, table_hbm, out_hbm, idx_v, rows_v, gsem, wsem):
    wid = lax.axis_index("s") * NUM_CORES + lax.axis_index("c")
    base = wid * PER_WORKER

    pltpu.sync_copy(idx_hbm.at[pl.ds(base, PER_WORKER)], idx_v)

    def start_gather(j, b):
        pltpu.async_copy(
            table_hbm.at[idx_v.at[pl.ds(j * CHUNK, CHUNK)]],
            rows_v.at[b],
            gsem.at[b],
        )

    def wait_gather(b):
        pltpu.make_async_copy(
            table_hbm.at[idx_v.at[pl.ds(0, CHUNK)]], rows_v.at[b], gsem.at[b]
        ).wait()

    def start_write(j, b):
        pltpu.async_copy(
            rows_v.at[b], out_hbm.at[pl.ds(base + j * CHUNK, CHUNK)], wsem.at[b]
        )

    def wait_write(b):
        pltpu.make_async_copy(
            rows_v.at[b], out_hbm.at[pl.ds(base, CHUNK)], wsem.at[b]
        ).wait()

    for b in range(NBUF):
        start_gather(b, b)

    @pl.loop(0, NCHUNKS - NBUF, step=NBUF)
    def _main(j0):
        for b in range(NBUF):
            j = j0 + b
            wait_gather(b)
            start_write(j, b)
            wait_write(b)
            start_gather(j + NBUF, b)

    for b in range(NBUF):
        wait_gather(b)
        start_write(NCHUNKS - NBUF + b, b)
        wait_write(b)


@functools.partial(jax.jit, static_argnames=())
def _sc_gather(idx_flat, table):
    mesh = plsc.VectorSubcoreMesh(
        core_axis_name="c", subcore_axis_name="s",
        num_cores=NUM_CORES, num_subcores=NUM_SUBCORES,
    )
    call = pl.kernel(
        _sc_body,
        out_type=jax.ShapeDtypeStruct((ROWS, DIM), jnp.float32),
        mesh=mesh,
        scratch_types=[
            pltpu.VMEM((PER_WORKER,), jnp.int32),
            pltpu.VMEM((NBUF, CHUNK, DIM), jnp.float32),
            pltpu.SemaphoreType.DMA((NBUF,)),
            pltpu.SemaphoreType.DMA((NBUF,)),
        ],
    )
    return call(idx_flat, table)


def kernel(indices, embeddings):
    idx_flat = indices.reshape(ROWS).astype(jnp.int32)
    out = _sc_gather(idx_flat, embeddings)
    return out.reshape(BATCH, TOKENS, DIM)


# speedup vs baseline: 1.6010x; 1.0023x over previous
import functools

import jax
import jax.numpy as jnp
from jax import lax
from jax.experimental import pallas as pl
from jax.experimental.pallas import tpu as pltpu
from jax.experimental.pallas import tpu_sc as plsc

BATCH = 4096
TOKENS = 200
DIM = 2048
ROWS = BATCH * TOKENS

NUM_CORES = 2
NUM_SUBCORES = 16
NUM_WORKERS = NUM_CORES * NUM_SUBCORES
PER_WORKER = ROWS // NUM_WORKERS

CHUNK = 8
NBUF = 4
NCHUNKS = PER_WORKER // CHUNK


def _sc_body(idx_hbm, table_hbm, out_hbm, idx_v, rows_v, gsem, wsem):
    wid = lax.axis_index("s") * NUM_CORES + lax.axis_index("c")
    base = wid * PER_WORKER

    pltpu.sync_copy(idx_hbm.at[pl.ds(base, PER_WORKER)], idx_v)

    def start_gather(j, b):
        pltpu.async_copy(
            table_hbm.at[idx_v.at[pl.ds(j * CHUNK, CHUNK)]],
            rows_v.at[b],
            gsem.at[b],
        )

    def wait_gather(b):
        pltpu.make_async_copy(
            table_hbm.at[idx_v.at[pl.ds(0, CHUNK)]], rows_v.at[b], gsem.at[b]
        ).wait()

    def start_write(j, b):
        pltpu.async_copy(
            rows_v.at[b], out_hbm.at[pl.ds(base + j * CHUNK, CHUNK)], wsem.at[b]
        )

    def wait_write(b):
        pltpu.make_async_copy(
            rows_v.at[b], out_hbm.at[pl.ds(base, CHUNK)], wsem.at[b]
        ).wait()

    for b in range(NBUF):
        start_gather(b, b)

    @pl.loop(0, NCHUNKS - NBUF, step=NBUF)
    def _main(j0):
        for b in range(NBUF):
            j = j0 + b
            wait_gather(b)
            start_write(j, b)
            wait_write(b)
            start_gather(j + NBUF, b)

    for b in range(NBUF):
        wait_gather(b)
        start_write(NCHUNKS - NBUF + b, b)
        wait_write(b)


@functools.partial(jax.jit, static_argnames=())
def _sc_gather(idx_flat, table):
    mesh = plsc.VectorSubcoreMesh(
        core_axis_name="c", subcore_axis_name="s",
        num_cores=NUM_CORES, num_subcores=NUM_SUBCORES,
    )
    call = pl.kernel(
        _sc_body,
        out_type=jax.ShapeDtypeStruct((ROWS, DIM), jnp.float32),
        mesh=mesh,
        scratch_types=[
            pltpu.VMEM((PER_WORKER,), jnp.int32),
            pltpu.VMEM((NBUF, CHUNK, DIM), jnp.float32),
            pltpu.SemaphoreType.DMA((NBUF,)),
            pltpu.SemaphoreType.DMA((NBUF,)),
        ],
    )
    return call(idx_flat, table)


def kernel(indices, embeddings):
    idx_flat = indices.reshape(ROWS).astype(jnp.int32)
    out = _sc_gather(idx_flat, embeddings)
    return out.reshape(BATCH, TOKENS, DIM)
